# baseline (speedup 1.0000x reference)
import dataclasses

import jax
import jax.numpy as jnp
import numpy as np
from jax import lax
from jax.experimental import pallas as pl
from jax.experimental.pallas import tpu as pltpu
from jax.experimental.pallas import tpu_sc as plsc

N = 256
E = 8192
K = 8
NORB = 4
ROWS = N * NORB
NPS = 16
FCOLS = NPS * NORB * 2
ARH = ROWS // 2
SEG = 1024
CAP = 512
CH = 128


def _expand_mats():
    m = np.zeros((13, 16), np.float32)
    m[0, 0] = 0.5
    for j in range(3):
        m[1 + j, 1 + j] = 1.0
    for i in range(3):
        for j in range(3):
            m[4 + 3 * i + j, (1 + i) * 4 + 1 + j] = 0.5
    m2 = np.zeros((13, 16), np.float32)
    for a in range(4):
        for b in range(4):
            m2[:, a * 4 + b] = m[:, a * 4 + b] + m[:, b * 4 + a]
    return jnp.asarray(m), jnp.asarray(m2)


def _tc_prep_body(ef_ref, nf_ref, kp_ref, rs_ref, m_ref, m2_ref, hp_o, ons_o):
    dn = (((1,), (0,)), ((), ()))
    hp_o[:, 0:16] = lax.dot_general(ef_ref[...], m_ref[...], dn,
                                    preferred_element_type=jnp.float32, precision=lax.Precision.HIGHEST)
    theta = 2.0 * jnp.pi * lax.dot_general(
        rs_ref[...], kp_ref[...], (((1,), (1,)), ((), ())),
        preferred_element_type=jnp.float32)
    hp_o[:, 16:24] = jnp.cos(theta)
    hp_o[:, 24:32] = -jnp.sin(theta)
    ons_o[...] = lax.dot_general(nf_ref[...], m2_ref[...], dn,
                                 preferred_element_type=jnp.float32, precision=lax.Precision.HIGHEST)


def _tc_prep(edge_features, node_features, kpoints, edge_cell_shift):
    m, m2 = _expand_mats()
    return pl.pallas_call(
        _tc_prep_body,
        out_shape=[
            jax.ShapeDtypeStruct((E, 32), jnp.float32),
            jax.ShapeDtypeStruct((N, 16), jnp.float32),
        ],
    )(edge_features, node_features, kpoints, edge_cell_shift, m, m2)


def _dyn_lane(vec, idx16):
    return lax.gather(
        vec, idx16[:, None],
        lax.GatherDimensionNumbers(offset_dims=(), collapsed_slice_dims=(0,),
                                   start_index_map=(0,)),
        slice_sizes=(1,), mode=lax.GatherScatterMode.PROMISE_IN_BOUNDS)


def _sc_body(hp_hbm, ons_hbm, src_hbm, dst_hbm, out_hbm,
             acc, sseg, dseg, bd_v, bh_v, hpd, hph, hps, cidx, onsv):
    rh = lax.axis_index("c")
    cs = lax.axis_index("s")
    lo = cs * NPS
    orow0 = rh * ARH

    lane = jnp.arange(16, dtype=jnp.int32)
    af = lane >> 2
    bf = lane & 3
    af2 = af * 2
    bf2 = bf * 2
    zf = jnp.zeros((16,), jnp.float32)
    zi = jnp.zeros((16,), jnp.int32)

    pltpu.sync_copy(ons_hbm.at[pl.ds(lo, NPS)], onsv)

    def zlists(g, carry):
        bd_v[pl.ds(g * 16, 16)] = zi
        bh_v[pl.ds(g * 16, 16)] = zi
        return carry

    lax.fori_loop(0, (E + 16) // 16, zlists, 0)

    def seg_pass(s, carry):
        pltpu.sync_copy(src_hbm.at[pl.ds(s * SEG, SEG)], sseg)
        pltpu.sync_copy(dst_hbm.at[pl.ds(s * SEG, SEG)], dseg)

        def build(g, inner):
            pd, ph_ = inner
            s16 = sseg[pl.ds(g * 16, 16)]
            d16 = dseg[pl.ds(g * 16, 16)]
            e16 = (s * SEG + g * 16) + lane
            dloc = d16 - lo
            sloc = s16 - lo
            srow = s16 * 4 - orow0
            drow = d16 * 4 - orow0
            md = (dloc >= 0) & (dloc < NPS) & (srow >= 0) & (srow < ARH)
            mh = (sloc >= 0) & (sloc < NPS) & (drow >= 0) & (drow < ARH)
            mdi = md.astype(jnp.int32)
            mhi = mh.astype(jnp.int32)
            posd = pd + jnp.cumsum(mdi) - 1
            posh = ph_ + jnp.cumsum(mhi) - 1
            plsc.store_scatter(bd_v, [posd],
                               (e16 << 17) | (srow << 7) | (dloc * 8),
                               mask=md)
            plsc.store_scatter(bh_v, [posh],
                               (e16 << 17) | (drow << 7) | (sloc * 8),
                               mask=mh)
            return pd + jnp.sum(mdi), ph_ + jnp.sum(mhi)

        return lax.fori_loop(0, SEG // 16, build, carry)

    nd, nh = lax.fori_loop(0, E // SEG, seg_pass,
                           (jnp.int32(0), jnp.int32(0)))

    dummy = jnp.full((16,), ARH << 7, jnp.int32)
    plsc.store_scatter(bd_v, [nd + lane], dummy)
    plsc.store_scatter(bh_v, [nh + lane], dummy)
    ndp = ((nd + 15) // 16) * 16
    nhp = ((nh + 15) // 16) * 16

    def cache(base_v, cache_ref, n_cached):
        def chunk(j, carry):
            c0 = j * CH

            def mkidx(g, cc):
                cidx[pl.ds(g * 16, 16)] = \
                    base_v[pl.ds(c0 + g * 16, 16)] >> 17
                return cc

            lax.fori_loop(0, CH // 16, mkidx, 0)
            pltpu.sync_copy(hp_hbm.at[cidx],
                            cache_ref.at[pl.ds(c0, CH)])
            return carry

        lax.fori_loop(0, (n_cached + CH - 1) // CH, chunk, 0)

    ncd = jnp.minimum(ndp, CAP)
    nch = jnp.minimum(nhp, CAP)
    cache(bd_v, hpd, ncd)
    cache(bh_v, hph, nch)

    def edge16(bv, data_ref, i_base, g, kre, kim, hermitian):
        for u in range(16):
            b = bv[u]
            rb = (b >> 7) & 1023
            cb = b & 127
            i = i_base + g * 16 + u
            hv = data_ref[i, pl.ds(0, 16)]
            phv = data_ref[i, pl.ds(16, 16)]
            vre = hv * _dyn_lane(phv, kre)
            vim = hv * _dyn_lane(phv, kim)
            if hermitian:
                rowv = rb + bf
                colv = cb + af2
                vim = -vim
            else:
                rowv = rb + af
                colv = cb + bf2
            plsc.addupdate_scatter(acc, [rowv, colv], vre)
            plsc.addupdate_scatter(acc, [rowv, colv + 1], vim)

    def groups(base_v, data_ref, count, kre, kim, hermitian):
        def group(g, gc):
            bv = base_v[pl.ds(g * 16, 16)]
            edge16(bv, data_ref, 0, g, kre, kim, hermitian)
            return gc

        lax.fori_loop(0, count // 16, group, 0)

    def spill(base_v, n_total, n_cached, kre, kim, hermitian):
        def chunk(j, carry):
            c0 = n_cached + j * CH

            def mkidx(g, cc):
                cidx[pl.ds(g * 16, 16)] = \
                    base_v[pl.ds(c0 + g * 16, 16)] >> 17
                return cc

            lax.fori_loop(0, CH // 16, mkidx, 0)
            pltpu.sync_copy(hp_hbm.at[cidx], hps)
            cnt = jnp.minimum(CH, n_total - c0)

            def group(g, gc):
                bv = base_v[pl.ds(c0 + g * 16, 16)]
                edge16(bv, hps, 0, g, kre, kim, hermitian)
                return gc

            lax.fori_loop(0, cnt // 16, group, 0)
            return carry

        lax.fori_loop(0, (n_total - n_cached + CH - 1) // CH, chunk, 0)

    def perk(kk, carry):
        kre = jnp.full((16,), 0, jnp.int32) + kk
        kim = kre + 8

        def zacc(r2, zc):
            for rr in range(2):
                for cc in range(8):
                    acc[r2 * 2 + rr, pl.ds(cc * 16, 16)] = zf
            return zc

        lax.fori_loop(0, ARH // 2, zacc, 0)

        for il in range(NPS):
            node = lo + il
            rbase = jnp.where((node * 4 >= orow0) & (node * 4 < orow0 + ARH),
                              node * 4 - orow0, ARH)
            rowv = rbase + af
            colv = il * 8 + bf2
            plsc.addupdate_scatter(acc, [rowv, colv], onsv[il, :])

        groups(bd_v, hpd, ncd, kre, kim, hermitian=False)
        spill(bd_v, ndp, ncd, kre, kim, hermitian=False)
        groups(bh_v, hph, nch, kre, kim, hermitian=True)
        spill(bh_v, nhp, nch, kre, kim, hermitian=True)

        pltpu.sync_copy(acc.at[pl.ds(0, ARH)],
                        out_hbm.at[kk, pl.ds(orow0, ARH),
                                   pl.ds(cs * FCOLS, FCOLS)])
        return carry

    lax.fori_loop(0, K, perk, 0)


def _sc_scatter(hp, ons16, src, dst):
    mesh = plsc.VectorSubcoreMesh(core_axis_name="c", subcore_axis_name="s")
    cp = pltpu.CompilerParams(use_tc_tiling_on_sc=False)
    if "needs_layout_passes" in pltpu.CompilerParams.__dataclass_fields__:
        cp = dataclasses.replace(cp, needs_layout_passes=False)
    kern = pl.kernel(
        _sc_body,
        out_type=jax.ShapeDtypeStruct((K, ROWS, 2 * ROWS), jnp.float32),
        mesh=mesh,
        compiler_params=cp,
        scratch_types=[
            pltpu.VMEM((ARH + 8, FCOLS), jnp.float32),
            pltpu.VMEM((SEG,), jnp.int32),
            pltpu.VMEM((SEG,), jnp.int32),
            pltpu.VMEM((E + 16,), jnp.int32),
            pltpu.VMEM((E + 16,), jnp.int32),
            pltpu.VMEM((CAP, 32), jnp.float32),
            pltpu.VMEM((CAP, 32), jnp.float32),
            pltpu.VMEM((CH, 32), jnp.float32),
            pltpu.VMEM((CH,), jnp.int32),
            pltpu.VMEM((NPS, 16), jnp.float32),
        ],
    )
    return kern(hp, ons16, src, dst)


def kernel(edge_features, node_features, atom_type, kpoints, edge_index,
           edge_cell_shift):
    del atom_type
    hp, ons16 = _tc_prep(
        edge_features.astype(jnp.float32),
        node_features.astype(jnp.float32),
        kpoints.astype(jnp.float32),
        edge_cell_shift.astype(jnp.float32))
    src = edge_index[0].astype(jnp.int32)
    dst = edge_index[1].astype(jnp.int32)
    outf = _sc_scatter(hp, ons16, src, dst)
    return lax.complex(outf[:, :, 0::2], outf[:, :, 1::2])


# speedup vs baseline: 1.0191x; 1.0191x over previous
import dataclasses

import jax
import jax.numpy as jnp
import numpy as np
from jax import lax
from jax.experimental import pallas as pl
from jax.experimental.pallas import tpu as pltpu
from jax.experimental.pallas import tpu_sc as plsc

N = 256
E = 8192
K = 8
NORB = 4
ROWS = N * NORB
OCOLS = 2 * ROWS
RPT = 32
NPT = RPT // NORB
SEG = 1024
CAP = 512
CH = 64


def _expand_mats():
    m = np.zeros((13, 16), np.float32)
    m[0, 0] = 0.5
    for j in range(3):
        m[1 + j, 1 + j] = 1.0
    for i in range(3):
        for j in range(3):
            m[4 + 3 * i + j, (1 + i) * 4 + 1 + j] = 0.5
    m2 = np.zeros((13, 16), np.float32)
    for a in range(4):
        for b in range(4):
            m2[:, a * 4 + b] = m[:, a * 4 + b] + m[:, b * 4 + a]
    return jnp.asarray(m), jnp.asarray(m2)


def _tc_prep_body(ef_ref, nf_ref, kp_ref, rs_ref, m_ref, m2_ref, hp_o, ons_o):
    dn = (((1,), (0,)), ((), ()))
    hp_o[:, 0:16] = lax.dot_general(ef_ref[...], m_ref[...], dn,
                                    preferred_element_type=jnp.float32,
                                    precision=lax.Precision.HIGHEST)
    theta = 2.0 * jnp.pi * lax.dot_general(
        rs_ref[...], kp_ref[...], (((1,), (1,)), ((), ())),
        preferred_element_type=jnp.float32)
    hp_o[:, 16:24] = jnp.cos(theta)
    hp_o[:, 24:32] = -jnp.sin(theta)
    ons_o[...] = lax.dot_general(nf_ref[...], m2_ref[...], dn,
                                 preferred_element_type=jnp.float32,
                                 precision=lax.Precision.HIGHEST)


def _tc_prep(edge_features, node_features, kpoints, edge_cell_shift):
    m, m2 = _expand_mats()
    return pl.pallas_call(
        _tc_prep_body,
        out_shape=[
            jax.ShapeDtypeStruct((E, 32), jnp.float32),
            jax.ShapeDtypeStruct((N, 16), jnp.float32),
        ],
    )(edge_features, node_features, kpoints, edge_cell_shift, m, m2)


def _dyn_lane(vec, idx16):
    return lax.gather(
        vec, idx16[:, None],
        lax.GatherDimensionNumbers(offset_dims=(), collapsed_slice_dims=(0,),
                                   start_index_map=(0,)),
        slice_sizes=(1,), mode=lax.GatherScatterMode.PROMISE_IN_BOUNDS)


def _sc_body(hp_hbm, ons_hbm, src_hbm, dst_hbm, out_hbm,
             acc, sseg, dseg, bd_v, bh_v, hpd, hph, hps, hpsp, cidx, onsv):
    wid = lax.axis_index("c") * 16 + lax.axis_index("s")
    lo = wid * NPT
    lane = jnp.arange(16, dtype=jnp.int32)
    af = lane >> 2
    bf = lane & 3
    af2 = af * 2
    bf2 = bf * 2
    zf = jnp.zeros((16,), jnp.float32)
    zi = jnp.zeros((16,), jnp.int32)

    pltpu.sync_copy(ons_hbm.at[pl.ds(lo, NPT)], onsv)

    def zlists(g, carry):
        bd_v[pl.ds(g * 16, 16)] = zi
        bh_v[pl.ds(g * 16, 16)] = zi
        return carry

    lax.fori_loop(0, (E + 16) // 16, zlists, 0)

    def seg_pass(s, carry):
        pltpu.sync_copy(src_hbm.at[pl.ds(s * SEG, SEG)], sseg)
        pltpu.sync_copy(dst_hbm.at[pl.ds(s * SEG, SEG)], dseg)

        def build(g, inner):
            pd, ph_ = inner
            s16 = sseg[pl.ds(g * 16, 16)]
            d16 = dseg[pl.ds(g * 16, 16)]
            e16 = (s * SEG + g * 16) + lane
            sloc = s16 - lo
            dloc = d16 - lo
            md = (sloc >= 0) & (sloc < NPT)
            mh = (dloc >= 0) & (dloc < NPT)
            mdi = md.astype(jnp.int32)
            mhi = mh.astype(jnp.int32)
            posd = pd + jnp.cumsum(mdi) - 1
            posh = ph_ + jnp.cumsum(mhi) - 1
            plsc.store_scatter(bd_v, [posd],
                               (e16 << 17) | ((sloc * 4) << 11) | (d16 * 8),
                               mask=md)
            plsc.store_scatter(bh_v, [posh],
                               (e16 << 17) | ((dloc * 4) << 11) | (s16 * 8),
                               mask=mh)
            return pd + jnp.sum(mdi), ph_ + jnp.sum(mhi)

        return lax.fori_loop(0, SEG // 16, build, carry)

    nd, nh = lax.fori_loop(0, E // SEG, seg_pass,
                           (jnp.int32(0), jnp.int32(0)))

    dummy = jnp.full((16,), E << 17, jnp.int32)
    plsc.store_scatter(bd_v, [nd + lane], dummy)
    plsc.store_scatter(bh_v, [nh + lane], dummy)
    ndp = ((nd + 15) // 16) * 16
    nhp = ((nh + 15) // 16) * 16

    def fetch_chunk(base_v, c0):
        def mkidx(g, cc):
            cidx[pl.ds(g * 16, 16)] = base_v[pl.ds(c0 + g * 16, 16)] >> 19
            return cc

        lax.fori_loop(0, CH // 16, mkidx, 0)
        pltpu.sync_copy(hp_hbm.at[cidx], hps)

    def compact_chunk(base_v, c0, dst_ref, d0):
        def cgroup(g, cc):
            bv = base_v[pl.ds(c0 + g * 16, 16)]
            for u in range(16):
                sub32 = (bv[u] >> 12) & 96
                i = g * 16 + u
                dst_ref[pl.ds((d0 + i) * 32, 16)] = hps[i, pl.ds(sub32, 16)]
                dst_ref[pl.ds((d0 + i) * 32 + 16, 16)] = \
                    hps[i, pl.ds(sub32 + 16, 16)]
            return cc

        lax.fori_loop(0, CH // 16, cgroup, 0)

    def cache(base_v, cache_ref, n_cached):
        def chunk(j, carry):
            c0 = j * CH
            fetch_chunk(base_v, c0)
            compact_chunk(base_v, c0, cache_ref, c0)
            return carry

        lax.fori_loop(0, (n_cached + CH - 1) // CH, chunk, 0)

    ncd = jnp.minimum(ndp, CAP)
    nch = jnp.minimum(nhp, CAP)
    cache(bd_v, hpd, ncd)
    cache(bh_v, hph, nch)

    def edge16(bv, data_ref, i_base, g, kre, kim, hermitian):
        for u in range(16):
            b = bv[u]
            rb = (b >> 11) & 63
            cb = b & 2047
            i = i_base + g * 16 + u
            hv = data_ref[pl.ds(i * 32, 16)]
            phv = data_ref[pl.ds(i * 32 + 16, 16)]
            vre = hv * _dyn_lane(phv, kre)
            vim = hv * _dyn_lane(phv, kim)
            if hermitian:
                rowv = rb + bf
                colv = cb + af2
                vim = -vim
            else:
                rowv = rb + af
                colv = cb + bf2
            plsc.addupdate_scatter(acc, [rowv, colv], vre)
            plsc.addupdate_scatter(acc, [rowv, colv + 1], vim)

    def groups(base_v, data_ref, count, kre, kim, hermitian):
        def group(g, gc):
            bv = base_v[pl.ds(g * 16, 16)]
            edge16(bv, data_ref, 0, g, kre, kim, hermitian)
            return gc

        lax.fori_loop(0, count // 16, group, 0)

    def spill(base_v, n_total, n_cached, kre, kim, hermitian):
        def chunk(j, carry):
            c0 = n_cached + j * CH
            fetch_chunk(base_v, c0)
            compact_chunk(base_v, c0, hpsp, 0)
            cnt = jnp.minimum(CH, n_total - c0)

            def group(g, gc):
                bv = base_v[pl.ds(c0 + g * 16, 16)]
                edge16(bv, hpsp, 0, g, kre, kim, hermitian)
                return gc

            lax.fori_loop(0, cnt // 16, group, 0)
            return carry

        lax.fori_loop(0, (n_total - n_cached + CH - 1) // CH, chunk, 0)

    def perk(kk, carry):
        kre = jnp.full((16,), 0, jnp.int32) + kk
        kim = kre + 8

        def zacc(j, zc):
            r = j >> 3
            c0 = (j & 7) * 256
            for u in range(16):
                acc[r, pl.ds(c0 + u * 16, 16)] = zf
            return zc

        lax.fori_loop(0, RPT * 8, zacc, 0)

        for il in range(NPT):
            rowv = il * 4 + af
            colv = (lo + il) * 8 + bf2
            plsc.addupdate_scatter(acc, [rowv, colv], onsv[il, :])

        groups(bd_v, hpd, ncd, kre, kim, hermitian=False)
        spill(bd_v, ndp, ncd, kre, kim, hermitian=False)
        groups(bh_v, hph, nch, kre, kim, hermitian=True)
        spill(bh_v, nhp, nch, kre, kim, hermitian=True)

        pltpu.sync_copy(acc.at[pl.ds(0, RPT)],
                        out_hbm.at[kk, pl.ds(wid * RPT, RPT), :])
        return carry

    lax.fori_loop(0, K, perk, 0)


def _sc_scatter(hp, ons16, src, dst):
    mesh = plsc.VectorSubcoreMesh(core_axis_name="c", subcore_axis_name="s")
    cp = pltpu.CompilerParams()
    if "needs_layout_passes" in pltpu.CompilerParams.__dataclass_fields__:
        cp = dataclasses.replace(cp, needs_layout_passes=False)
    kern = pl.kernel(
        _sc_body,
        out_type=jax.ShapeDtypeStruct((K, ROWS, OCOLS), jnp.float32),
        mesh=mesh,
        compiler_params=cp,
        scratch_types=[
            pltpu.VMEM((RPT, OCOLS), jnp.float32),
            pltpu.VMEM((SEG,), jnp.int32),
            pltpu.VMEM((SEG,), jnp.int32),
            pltpu.VMEM((E + 16,), jnp.int32),
            pltpu.VMEM((E + 16,), jnp.int32),
            pltpu.VMEM((CAP * 32,), jnp.float32),
            pltpu.VMEM((CAP * 32,), jnp.float32),
            pltpu.VMEM((CH, 128), jnp.float32),
            pltpu.VMEM((CH * 32,), jnp.float32),
            pltpu.VMEM((CH,), jnp.int32),
            pltpu.VMEM((NPT, 16), jnp.float32),
        ],
    )
    return kern(hp, ons16, src, dst)


def kernel(edge_features, node_features, atom_type, kpoints, edge_index,
           edge_cell_shift):
    del atom_type
    hp, ons16 = _tc_prep(
        edge_features.astype(jnp.float32),
        node_features.astype(jnp.float32),
        kpoints.astype(jnp.float32),
        edge_cell_shift.astype(jnp.float32))
    hp = jnp.concatenate(
        [hp.reshape(E // 4, 128), jnp.zeros((8, 128), jnp.float32)])
    src = edge_index[0].astype(jnp.int32)
    dst = edge_index[1].astype(jnp.int32)
    outf = _sc_scatter(hp, ons16, src, dst)
    return lax.complex(outf[:, :, 0::2], outf[:, :, 1::2])
